# Initial kernel scaffold; baseline (speedup 1.0000x reference)
import functools

import jax
import jax.numpy as jnp
from jax import lax
from jax.experimental import pallas as pl
from jax.experimental.pallas import tpu as pltpu
from jax.experimental.pallas import tpu_sc as plsc

N_NODES = 10000
N_EDGES = 320000
D_FEAT = 128
LANES = 16

NC = 2
NS = 16
NW = NC * NS

EDGES_PER_TILE = N_EDGES // NW
CHUNK = 200
NCHUNK = EDGES_PER_TILE // CHUNK


def _body(src_hbm, dst_hbm, model_hbm, out_hbm,
          sidx, didx, srows, drows, outv, sem):
    wid = lax.axis_index("s") * NC + lax.axis_index("c")
    base = wid * EDGES_PER_TILE

    def chunk_step(j, _):
        off = base + j * CHUNK
        pltpu.sync_copy(src_hbm.at[pl.ds(off, CHUNK)], sidx)
        pltpu.sync_copy(dst_hbm.at[pl.ds(off, CHUNK)], didx)
        g0 = pltpu.async_copy(model_hbm.at[sidx], srows, sem)
        g1 = pltpu.async_copy(model_hbm.at[didx], drows, sem)
        g0.wait()
        g1.wait()

        def edge_step(e, _):
            acc = srows[e, pl.ds(0, LANES)] * drows[e, pl.ds(0, LANES)]
            for k in range(1, D_FEAT // LANES):
                acc += (srows[e, pl.ds(k * LANES, LANES)]
                        * drows[e, pl.ds(k * LANES, LANES)])
            outv[e] = jnp.sum(acc)
            return 0

        lax.fori_loop(0, CHUNK, edge_step, 0)
        pltpu.sync_copy(outv, out_hbm.at[pl.ds(off, CHUNK)])
        return 0

    lax.fori_loop(0, NCHUNK, chunk_step, 0)


@jax.jit
def _run(src, dst, model):
    mesh = plsc.VectorSubcoreMesh(core_axis_name="c", subcore_axis_name="s")
    return pl.kernel(
        _body,
        out_type=jax.ShapeDtypeStruct((N_EDGES,), jnp.float32),
        mesh=mesh,
        scratch_types=[
            pltpu.VMEM((CHUNK,), jnp.int32),
            pltpu.VMEM((CHUNK,), jnp.int32),
            pltpu.VMEM((CHUNK, D_FEAT), jnp.float32),
            pltpu.VMEM((CHUNK, D_FEAT), jnp.float32),
            pltpu.VMEM((CHUNK,), jnp.float32),
            pltpu.SemaphoreType.DMA,
        ],
    )(src, dst, model)


def kernel(model, edge_index):
    ei = edge_index.astype(jnp.int32)
    return _run(ei[0], ei[1], model)


# speedup vs baseline: 1.2078x; 1.2078x over previous
import functools

import jax
import jax.numpy as jnp
from jax import lax
from jax.experimental import pallas as pl
from jax.experimental.pallas import tpu as pltpu
from jax.experimental.pallas import tpu_sc as plsc

N_NODES = 10000
N_EDGES = 320000
D_FEAT = 128
LANES = 16

NC = 2
NS = 16
NW = NC * NS

EDGES_PER_TILE = N_EDGES // NW
CHUNK = 400
NCHUNK = EDGES_PER_TILE // CHUNK
NGROUP = CHUNK // LANES


def _body(src_hbm, dst_hbm, model_hbm, out_hbm,
          sidx, didx, srows, drows, outv, sem):
    wid = lax.axis_index("s") * NC + lax.axis_index("c")
    base = wid * EDGES_PER_TILE

    def chunk_step(j, _):
        off = base + j * CHUNK
        pltpu.sync_copy(src_hbm.at[pl.ds(off, CHUNK)], sidx)
        pltpu.sync_copy(dst_hbm.at[pl.ds(off, CHUNK)], didx)
        g0 = pltpu.async_copy(model_hbm.at[sidx], srows, sem)
        g1 = pltpu.async_copy(model_hbm.at[didx], drows, sem)
        g0.wait()
        g1.wait()

        lane = lax.iota(jnp.int32, LANES)

        def group_step(g, _):
            rows = g * LANES + lane
            acc = jnp.zeros((LANES,), jnp.float32)
            for d in range(D_FEAT):
                col = jnp.full((LANES,), d, jnp.int32)
                acc += (plsc.load_gather(srows, [rows, col])
                        * plsc.load_gather(drows, [rows, col]))
            outv[pl.ds(g * LANES, LANES)] = acc
            return 0

        lax.fori_loop(0, NGROUP, group_step, 0)
        pltpu.sync_copy(outv, out_hbm.at[pl.ds(off, CHUNK)])
        return 0

    lax.fori_loop(0, NCHUNK, chunk_step, 0)


@jax.jit
def _run(src, dst, model):
    mesh = plsc.VectorSubcoreMesh(core_axis_name="c", subcore_axis_name="s")
    return pl.kernel(
        _body,
        out_type=jax.ShapeDtypeStruct((N_EDGES,), jnp.float32),
        mesh=mesh,
        compiler_params=pltpu.CompilerParams(needs_layout_passes=False),
        scratch_types=[
            pltpu.VMEM((CHUNK,), jnp.int32),
            pltpu.VMEM((CHUNK,), jnp.int32),
            pltpu.VMEM((CHUNK, D_FEAT), jnp.float32),
            pltpu.VMEM((CHUNK, D_FEAT), jnp.float32),
            pltpu.VMEM((CHUNK,), jnp.float32),
            pltpu.SemaphoreType.DMA,
        ],
    )(src, dst, model)


def kernel(model, edge_index):
    ei = edge_index.astype(jnp.int32)
    return _run(ei[0], ei[1], model)
